# baseline (speedup 1.0000x reference)
import numpy as np
import jax
import jax.numpy as jnp
from jax import lax
from jax.experimental import pallas as pl
from jax.experimental.pallas import tpu as pltpu
from jax.experimental.pallas import tpu_sc as plsc

GRID = 128
PATCH = 8
IMG = GRID * PATCH
N_NODES = GRID * GRID
N_EDGES = N_NODES * 16
EPS = 1e-8

F32 = jnp.float32
I32 = jnp.int32
_HI = lax.Precision.HIGHEST


def _conv_selector(in_h, out_h):
    s = np.zeros((3, 3, out_h, out_h, in_h, in_h), np.float32)
    for di in range(3):
        for dj in range(3):
            for oi in range(out_h):
                for oj in range(out_h):
                    r, q = 2 * oi + di, 2 * oj + dj
                    if r < in_h and q < in_h:
                        s[di, dj, oi, oj, r, q] = 1.0
    return s

_S1F = _conv_selector(8, 4).transpose(2, 3, 4, 5, 0, 1).reshape(-1, 9).T.copy()
_S2F = _conv_selector(4, 2).transpose(2, 3, 4, 5, 0, 1).reshape(-1, 9).T.copy()

def _group_mat(n_feat, n_grp):
    g = np.zeros((n_feat, n_grp), np.float32)
    w = n_feat // n_grp
    for i in range(n_feat):
        g[i, i // w] = 1.0
    return g

_G1 = _group_mat(640, 16)
_G1T = _G1.T.copy()
_G2 = _group_mat(160, 4)
_G2T = _G2.T.copy()

_LS = np.zeros((GRID, IMG), np.float32)
for _h in range(IMG):
    _LS[_h // PATCH, _h] = 1.0
_RS = np.zeros((IMG * 4, GRID * 4), np.float32)
for _w in range(IMG):
    for _c in range(4):
        _RS[4 * _w + _c, (_w // PATCH) * 4 + _c] = 1.0

_SELT = np.zeros((8, 128), np.float32)
_SEL0 = np.zeros((128, 8), np.float32)
_SH1 = np.zeros((128, 128), np.float32)
_SLG = np.zeros((128, 16), np.float32)
for _k in range(8):
    _SELT[_k, 16 * _k] = 1.0
    _SEL0[16 * _k, _k] = 1.0
    _SH1[16 * _k + 1, 16 * _k] = 1.0
    _SLG[16 * _k, 2 * _k] = 1.0
    _SLG[16 * _k + 1, 2 * _k + 1] = 1.0

_MESH = dict(core_axis_name="c", subcore_axis_name="s",
             num_cores=2, num_subcores=16)
_NW = 32


_TPW = N_NODES // _NW
_IDROWS = 1024

def _tok_body(itab, mtab, ids, xout, idsv, ibuf, rbuf, xbuf, sem):
    c = lax.axis_index("c")
    sid = lax.axis_index("s")
    w = sid * 2 + c
    shift = w // 8
    sy = shift // 2
    col0 = 32 * sy + shift
    icol0 = 8 * sy
    pltpu.sync_copy(ids.at[pl.ds(w * 32, 32)], idsv)
    iota16 = lax.broadcasted_iota(I32, (16,), 0)

    def chunk(cidx, carry):
        cps = []
        for j in range(4):
            cps.append(pltpu.async_copy(mtab.at[idsv.at[cidx * 4 + j]],
                                        rbuf.at[j], sem))
            cps.append(pltpu.async_copy(itab.at[idsv.at[cidx * 4 + j]],
                                        ibuf.at[j], sem))
        for cp in cps:
            cp.wait()

        def tok(lt, carry2):
            blk = lt // 16
            lt16 = lt % 16
            half = lt & 1
            ja = jnp.full((16,), blk, I32)
            for k in range(4):
                flat = iota16 + (16 * k)
                rowi = lt16 * 8 + (flat >> 3)
                mval = plsc.load_gather(rbuf, [ja, rowi,
                                               half * 64 + col0
                                               + 4 * (flat & 7)])
                ival = plsc.load_gather(ibuf, [ja, rowi,
                                               half * 16 + icol0
                                               + (flat & 7)])
                xbuf[lt, pl.ds(16 * k, 16)] = ival * mval
            return carry2

        lax.fori_loop(0, 64, tok, 0)
        pltpu.sync_copy(xbuf, xout.at[pl.ds(w * _TPW + cidx * 64, 64)])
        return carry

    lax.fori_loop(0, 8, chunk, 0)


def _sc_tokenize(img_tab, msk_tab, ids2d):
    mesh = plsc.VectorSubcoreMesh(**_MESH)
    fn = pl.kernel(
        _tok_body,
        out_type=jax.ShapeDtypeStruct((N_NODES, 64), F32),
        mesh=mesh,
        scratch_types=[
            pltpu.VMEM((32, 128), I32),
            pltpu.VMEM((4, 128, 32), F32),
            pltpu.VMEM((4, 128, 128), F32),
            pltpu.VMEM((64, 64), F32),
            pltpu.SemaphoreType.DMA,
        ],
        compiler_params=pltpu.CompilerParams(use_tc_tiling_on_sc=False,
                                             needs_layout_passes=False),
    )
    return fn(img_tab, msk_tab, ids2d)


def _psum_body(m4, lm4, lsr, rsb, out):
    u1 = jnp.dot(lsr[...], m4[...], precision=_HI)
    s1 = jnp.dot(u1, rsb[0], precision=_HI)
    u2 = jnp.dot(lsr[...], lm4[...], precision=_HI)
    s2 = jnp.dot(u2, rsb[0], precision=_HI)
    out[0] = jnp.round(s2 / (s1 + EPS))


def _patch_labels(m4, lm4):
    rs3 = jnp.asarray(_RS.reshape(8, 512, 8, 64)[np.arange(8), :,
                                                 np.arange(8)])
    return pl.pallas_call(
        _psum_body,
        grid=(8,),
        in_specs=[
            pl.BlockSpec((IMG, 512), lambda b: (0, b)),
            pl.BlockSpec((IMG, 512), lambda b: (0, b)),
            pl.BlockSpec((GRID, IMG), lambda b: (0, 0)),
            pl.BlockSpec((1, 512, 64), lambda b: (b, 0, 0)),
        ],
        out_specs=pl.BlockSpec((1, GRID, 64), lambda b: (b, 0, 0)),
        out_shape=jax.ShapeDtypeStruct((8, GRID, 64), F32),
    )(m4, lm4, jnp.asarray(_LS), rs3)


_RB_A = 512
_NB_A = N_NODES // _RB_A

def _enc_body(x_ref, m1, b1r, m2, b2r, d1w, d1b, ws, wr, bs, br,
              g1_ref, g1t_ref, g2_ref, g2t_ref,
              t_out, tabs_out, tabr_out):
    g1 = g1_ref[...]
    g1t = g1t_ref[...]
    g2 = g2_ref[...]
    g2t = g2t_ref[...]
    x = x_ref[...]
    y0 = jnp.dot(x, m1[...], precision=_HI) + b1r[...]
    mu = jnp.dot(y0, g1, precision=_HI) * (1.0 / 40.0)
    d = y0 - jnp.dot(mu, g1t, precision=_HI)
    var = jnp.dot(d * d, g1, precision=_HI) * (1.0 / 40.0)
    y = d / jnp.sqrt(jnp.dot(var, g1t, precision=_HI) + 1e-6)
    y = jnp.maximum(y, 0.0)
    z0 = jnp.dot(y, m2[...], precision=_HI) + b2r[...]
    mu2 = jnp.dot(z0, g2, precision=_HI) * (1.0 / 40.0)
    d2 = z0 - jnp.dot(mu2, g2t, precision=_HI)
    var2 = jnp.dot(d2 * d2, g2, precision=_HI) * (1.0 / 40.0)
    z = d2 / jnp.sqrt(jnp.dot(var2, g2t, precision=_HI) + 1e-6)
    z = jnp.maximum(z, 0.0)
    t = jnp.maximum(jnp.dot(z, d1w[...], precision=_HI) + d1b[...], 0.0)
    t_out[...] = t
    tabs_out[...] = jnp.dot(t, ws[...], precision=_HI) + bs[...]
    tabr_out[...] = jnp.dot(t, wr[...], precision=_HI) + br[...]


def _encoder(x_tok, m1, b1r, m2, b2r, d1w, d1b, ws, wr, bs, br):
    row = lambda i: (i, 0)
    full = lambda i: (0, 0)
    return pl.pallas_call(
        _enc_body,
        grid=(_NB_A,),
        in_specs=[
            pl.BlockSpec((_RB_A, 64), row),
            pl.BlockSpec((64, 640), full),
            pl.BlockSpec((1, 640), full),
            pl.BlockSpec((640, 160), full),
            pl.BlockSpec((1, 160), full),
            pl.BlockSpec((160, 16), full),
            pl.BlockSpec((1, 16), full),
            pl.BlockSpec((16, 16), full),
            pl.BlockSpec((16, 16), full),
            pl.BlockSpec((1, 16), full),
            pl.BlockSpec((1, 16), full),
            pl.BlockSpec((640, 16), full),
            pl.BlockSpec((16, 640), full),
            pl.BlockSpec((160, 4), full),
            pl.BlockSpec((4, 160), full),
        ],
        out_specs=[
            pl.BlockSpec((_RB_A, 16), row),
            pl.BlockSpec((_RB_A, 16), row),
            pl.BlockSpec((_RB_A, 16), row),
        ],
        out_shape=[
            jax.ShapeDtypeStruct((N_NODES, 16), F32),
            jax.ShapeDtypeStruct((N_NODES, 16), F32),
            jax.ShapeDtypeStruct((N_NODES, 16), F32),
        ],
    )(x_tok, m1, b1r, m2, b2r, d1w, d1b, ws, wr, bs, br,
      jnp.asarray(_G1), jnp.asarray(_G1T), jnp.asarray(_G2), jnp.asarray(_G2T))


_EPW = N_EDGES // _NW
_CH = 128
_ROWS_PER_W = _EPW // _CH
_GRP = 8
_NROW_E = N_EDGES // 8

def _gather_body(snd, rcv, tabs, tabr, outs, outr,
                 idxs, idxr, bufs, bufr, sem):
    c = lax.axis_index("c")
    s = lax.axis_index("s")
    w = s * 2 + c
    base = w * _ROWS_PER_W
    pltpu.sync_copy(snd.at[pl.ds(base, _ROWS_PER_W)], idxs)
    pltpu.sync_copy(rcv.at[pl.ds(base, _ROWS_PER_W)], idxr)

    def step(k, carry):
        cps = []
        for j in range(_GRP):
            cps.append(pltpu.async_copy(tabs.at[idxs.at[k * _GRP + j]],
                                        bufs.at[j], sem))
            cps.append(pltpu.async_copy(tabr.at[idxr.at[k * _GRP + j]],
                                        bufr.at[j], sem))
        for cp in cps:
            cp.wait()
        pltpu.sync_copy(bufs, outs.at[pl.ds(base + k * _GRP, _GRP)])
        pltpu.sync_copy(bufr, outr.at[pl.ds(base + k * _GRP, _GRP)])
        return carry

    lax.fori_loop(0, _ROWS_PER_W // _GRP, step, 0)


def _edge_gather(snd2d, rcv2d, tabs, tabr):
    mesh = plsc.VectorSubcoreMesh(**_MESH)
    fn = pl.kernel(
        _gather_body,
        out_type=[
            jax.ShapeDtypeStruct((N_EDGES // _CH, _CH, 16), F32),
            jax.ShapeDtypeStruct((N_EDGES // _CH, _CH, 16), F32),
        ],
        mesh=mesh,
        scratch_types=[
            pltpu.VMEM((_ROWS_PER_W, _CH), I32),
            pltpu.VMEM((_ROWS_PER_W, _CH), I32),
            pltpu.VMEM((_GRP, _CH, 16), F32),
            pltpu.VMEM((_GRP, _CH, 16), F32),
            pltpu.SemaphoreType.DMA,
        ],
        compiler_params=pltpu.CompilerParams(use_tc_tiling_on_sc=False),
    )
    return fn(snd2d, rcv2d, tabs, tabr)


_RB_E = 4096
_NB_E = _NROW_E // _RB_E

def _emlp_body(srow, rrow, w2, b2, w3, b3, out):
    h = jnp.maximum(srow[...] + rrow[...], 0.0)
    h = jnp.maximum(jnp.dot(h, w2[...], precision=_HI) + b2[...], 0.0)
    h = jnp.maximum(jnp.dot(h, w3[...], precision=_HI) + b3[...], 0.0)
    out[...] = h


def _edge_mlp(srows, rrows, w2, b2, w3, b3):
    row = lambda i: (i, 0)
    full = lambda i: (0, 0)
    return pl.pallas_call(
        _emlp_body,
        grid=(_NB_E,),
        in_specs=[
            pl.BlockSpec((_RB_E, 128), row),
            pl.BlockSpec((_RB_E, 128), row),
            pl.BlockSpec((128, 128), full),
            pl.BlockSpec((1, 128), full),
            pl.BlockSpec((128, 128), full),
            pl.BlockSpec((1, 128), full),
        ],
        out_specs=pl.BlockSpec((_RB_E, 128), row),
        out_shape=jax.ShapeDtypeStruct((_NROW_E, 128), F32),
    )(srows, rrows, w2, b2, w3, b3)


_ZRB = N_NODES // 16

def _scatter_body(msg, rcv, zeros, out0, out1, agg, mbuf, ridx, sem):
    c = lax.axis_index("c")
    s = lax.axis_index("s")
    w = s * 2 + c
    pltpu.sync_copy(zeros.at[pl.ds(s * _ZRB, _ZRB)],
                    agg.at[pl.ds(s * _ZRB, _ZRB)])
    plsc.subcore_barrier()
    base = w * _ROWS_PER_W
    pltpu.sync_copy(rcv.at[pl.ds(base, _ROWS_PER_W)], ridx)

    def step(k, carry):
        cp = pltpu.async_copy(msg.at[pl.ds(base + k * _GRP, _GRP)], mbuf, sem)
        cp.wait()
        cps = []
        for j in range(_GRP):
            cps.append(pltpu.async_copy(mbuf.at[j],
                                        agg.at[ridx.at[k * _GRP + j]],
                                        sem, add=True))
        for cp2 in cps:
            cp2.wait()
        return carry

    lax.fori_loop(0, _ROWS_PER_W // _GRP, step, 0)
    plsc.subcore_barrier()

    @pl.when(c == 0)
    def _():
        pltpu.sync_copy(agg.at[pl.ds(s * _ZRB, _ZRB)],
                        out0.at[pl.ds(s * _ZRB, _ZRB)])

    @pl.when(c == 1)
    def _():
        pltpu.sync_copy(agg.at[pl.ds(s * _ZRB, _ZRB)],
                        out1.at[pl.ds(s * _ZRB, _ZRB)])


def _edge_scatter(msg2d, rcv2d, zeros):
    mesh = plsc.VectorSubcoreMesh(**_MESH)
    fn = pl.kernel(
        _scatter_body,
        out_type=[
            jax.ShapeDtypeStruct((N_NODES, 16), F32),
            jax.ShapeDtypeStruct((N_NODES, 16), F32),
        ],
        mesh=mesh,
        scratch_types=[
            pltpu.VMEM_SHARED((N_NODES, 16), F32),
            pltpu.VMEM((_GRP, _CH, 16), F32),
            pltpu.VMEM((_ROWS_PER_W, _CH), I32),
            pltpu.SemaphoreType.DMA,
        ],
        compiler_params=pltpu.CompilerParams(use_tc_tiling_on_sc=False),
    )
    return fn(msg2d, rcv2d, zeros)


_NROW_C = N_NODES // 8
_RB_C = 1024
_NB_C = _NROW_C // _RB_C

def _node_body(t, p0, p1, lab8, wt, wa, b1, w2, b2, w3, b3, d2w, d2b,
               selt, sel0, sh1, slg,
               loss_out, logit_out):
    agg = p0[...] + p1[...]
    nf = jnp.maximum(jnp.dot(t[...], wt[...], precision=_HI)
                     + jnp.dot(agg, wa[...], precision=_HI) + b1[...], 0.0)
    nf = jnp.maximum(jnp.dot(nf, w2[...], precision=_HI) + b2[...], 0.0)
    nf = jnp.maximum(jnp.dot(nf, w3[...], precision=_HI) + b3[...], 0.0)
    lg = jnp.dot(nf, d2w[...], precision=_HI) + d2b[...]
    l1 = jnp.dot(lg, sh1[...], precision=_HI)
    m = jnp.maximum(lg, l1)
    lse = m + jnp.log(jnp.exp(lg - m) + jnp.exp(l1 - m))
    lp0 = lg - lse
    lp1 = l1 - lse
    lab = jnp.dot(lab8[...], selt[...], precision=_HI)
    loss = -((1.0 - lab) * lp0 + lab * lp1)
    loss_out[...] = jnp.dot(loss, sel0[...], precision=_HI)
    logit_out[...] = jnp.dot(lg, slg[...], precision=_HI)


def _node_mlp(t, p0, p1, lab8, wt, wa, b1, w2, b2, w3, b3, d2w, d2b):
    row = lambda i: (i, 0)
    full = lambda i: (0, 0)
    return pl.pallas_call(
        _node_body,
        grid=(_NB_C,),
        in_specs=[
            pl.BlockSpec((_RB_C, 128), row),
            pl.BlockSpec((_RB_C, 128), row),
            pl.BlockSpec((_RB_C, 128), row),
            pl.BlockSpec((_RB_C, 8), row),
            pl.BlockSpec((128, 128), full),
            pl.BlockSpec((128, 128), full),
            pl.BlockSpec((1, 128), full),
            pl.BlockSpec((128, 128), full),
            pl.BlockSpec((1, 128), full),
            pl.BlockSpec((128, 128), full),
            pl.BlockSpec((1, 128), full),
            pl.BlockSpec((128, 128), full),
            pl.BlockSpec((1, 128), full),
            pl.BlockSpec((8, 128), full),
            pl.BlockSpec((128, 8), full),
            pl.BlockSpec((128, 128), full),
            pl.BlockSpec((128, 16), full),
        ],
        out_specs=[
            pl.BlockSpec((_RB_C, 8), row),
            pl.BlockSpec((_RB_C, 16), row),
        ],
        out_shape=[
            jax.ShapeDtypeStruct((_NROW_C, 8), F32),
            jax.ShapeDtypeStruct((_NROW_C, 16), F32),
        ],
    )(t, p0, p1, lab8, wt, wa, b1, w2, b2, w3, b3, d2w, d2b,
      jnp.asarray(_SELT), jnp.asarray(_SEL0), jnp.asarray(_SH1),
      jnp.asarray(_SLG))


def kernel(curr_image, curr_label, masks, edge_pairs, conv1_w, conv1_b,
           conv2_w, conv2_b, dense1_w, dense1_b, e_w1, e_b1, e_w2, e_b2,
           e_w3, e_b3, n_w1, n_b1, n_w2, n_b2, n_w3, n_b3,
           dense2_w, dense2_b):
    msk4 = masks[0]
    labm4 = curr_label[0] * msk4

    tokv = jnp.arange(N_NODES, dtype=I32)
    sft = tokv >> 12
    syv = sft >> 1
    sxv = sft & 1
    rem = tokv & 4095
    gxv = rem >> 6
    gyv = rem & 63
    ids = ((16 * gxv + 8 * sxv)[:, None]
           + jnp.arange(8, dtype=I32)[None, :]) * 32 + (gyv >> 1)[:, None]
    ids2d = ids.reshape(_IDROWS, 128)

    x_tok = _sc_tokenize(curr_image.reshape(IMG * 32, 32),
                         masks.reshape(IMG * 32, 128), ids2d)

    labp3 = _patch_labels(msk4.reshape(IMG, IMG * 4),
                          labm4.reshape(IMG, IMG * 4))
    labp = labp3.transpose(1, 0, 2).reshape(GRID, GRID * 4)
    lp = labp.reshape(64, 2, 64, 2, 4)
    lp = lp.transpose(3, 1, 0, 2, 4).reshape(4, 4096, 4)
    oh4 = jnp.asarray(np.eye(4, dtype=np.float32))[:, None, :]
    labv = jnp.sum(lp * oh4, axis=-1).reshape(N_NODES)
    labels2 = jnp.stack([1.0 - labv, labv], axis=-1)
    lab8 = labv.reshape(_NROW_C, 8)

    c1 = jnp.dot(jnp.asarray(_S1F).T, conv1_w.reshape(9, 40), precision=_HI)
    m1 = c1.reshape(4, 4, 8, 8, 40).transpose(2, 3, 0, 1, 4).reshape(64, 640)
    c2 = jnp.dot(jnp.asarray(_S2F).T, conv2_w.reshape(9, 40 * 40),
                 precision=_HI)
    m2 = (c2.reshape(2, 2, 4, 4, 40, 40)
          .transpose(2, 3, 4, 0, 1, 5).reshape(640, 160))
    b1_tiled = jnp.tile(conv1_b, 16)[None, :]
    b2_tiled = jnp.tile(conv2_b, 4)[None, :]
    d1w = jnp.pad(dense1_w, ((0, 0), (0, 10)))
    d1b = jnp.pad(dense1_b, (0, 10))[None, :]

    half_b1 = 0.5 * (e_b1 + e_w1[12])
    ws = jnp.pad(e_w1[0:6], ((0, 10), (0, 11)))
    wr = jnp.pad(e_w1[6:12], ((0, 10), (0, 11)))
    bs = jnp.pad(half_b1, (0, 11))[None, :]

    w2p = jnp.pad(e_w2, ((0, 11), (0, 11)))
    b2p = jnp.pad(e_b2, (0, 11))[None, :]
    w3p = jnp.pad(e_w3, ((0, 11), (0, 11)))
    b3p = jnp.pad(e_b3, (0, 11))[None, :]

    eye8 = jnp.asarray(np.eye(8, dtype=np.float32))
    w2bd = jnp.kron(eye8, w2p)
    w3bd = jnp.kron(eye8, w3p)
    b2bd = jnp.tile(b2p, (1, 8))
    b3bd = jnp.tile(b3p, (1, 8))

    wt = jnp.kron(eye8, jnp.pad(n_w1[0:6], ((0, 10), (0, 11))))
    wa = jnp.kron(eye8, jnp.pad(n_w1[6:11], ((0, 11), (0, 11))))
    nb1 = jnp.tile(jnp.pad(n_b1 + n_w1[11], (0, 11)), 8)[None, :]
    nw2 = jnp.kron(eye8, jnp.pad(n_w2, ((0, 11), (0, 11))))
    nb2 = jnp.tile(jnp.pad(n_b2, (0, 11)), 8)[None, :]
    nw3 = jnp.kron(eye8, jnp.pad(n_w3, ((0, 11), (0, 11))))
    nb3 = jnp.tile(jnp.pad(n_b3, (0, 11)), 8)[None, :]
    d2w = jnp.kron(eye8, jnp.pad(dense2_w, ((0, 11), (0, 14))))
    d2b = jnp.tile(jnp.pad(dense2_b, (0, 14)), 8)[None, :]

    t16, tabs, tabr = _encoder(
        x_tok, m1, b1_tiled, m2, b2_tiled, d1w, d1b, ws, wr, bs, bs)

    snd2d = edge_pairs[:, 0].reshape(N_EDGES // _CH, _CH)
    rcv2d = edge_pairs[:, 1].reshape(N_EDGES // _CH, _CH)

    srows, rrows = _edge_gather(snd2d, rcv2d, tabs, tabr)
    msg = _edge_mlp(srows.reshape(_NROW_E, 128), rrows.reshape(_NROW_E, 128),
                    w2bd, b2bd, w3bd, b3bd)

    zeros = jnp.zeros((N_NODES, 16), F32)
    p0, p1 = _edge_scatter(msg.reshape(N_EDGES // _CH, _CH, 16), rcv2d, zeros)

    loss8, logits16 = _node_mlp(
        t16.reshape(_NROW_C, 128), p0.reshape(_NROW_C, 128),
        p1.reshape(_NROW_C, 128), lab8,
        wt, wa, nb1, nw2, nb2, nw3, nb3, d2w, d2b)
    return (loss8.reshape(N_NODES), logits16.reshape(N_NODES, 2), labels2)


# speedup vs baseline: 1.0814x; 1.0814x over previous
import numpy as np
import jax
import jax.numpy as jnp
from jax import lax
from jax.experimental import pallas as pl
from jax.experimental.pallas import tpu as pltpu
from jax.experimental.pallas import tpu_sc as plsc

GRID = 128
PATCH = 8
IMG = GRID * PATCH
N_NODES = GRID * GRID
N_EDGES = N_NODES * 16
EPS = 1e-8

F32 = jnp.float32
I32 = jnp.int32
_HI = lax.Precision.HIGHEST


def _conv_selector(in_h, out_h):
    s = np.zeros((3, 3, out_h, out_h, in_h, in_h), np.float32)
    for di in range(3):
        for dj in range(3):
            for oi in range(out_h):
                for oj in range(out_h):
                    r, q = 2 * oi + di, 2 * oj + dj
                    if r < in_h and q < in_h:
                        s[di, dj, oi, oj, r, q] = 1.0
    return s

_S1F = _conv_selector(8, 4).transpose(2, 3, 4, 5, 0, 1).reshape(-1, 9).T.copy()
_S2F = _conv_selector(4, 2).transpose(2, 3, 4, 5, 0, 1).reshape(-1, 9).T.copy()

def _group_mat(n_feat, n_grp):
    g = np.zeros((n_feat, n_grp), np.float32)
    w = n_feat // n_grp
    for i in range(n_feat):
        g[i, i // w] = 1.0
    return g

_G1 = _group_mat(640, 16)
_G1T = _G1.T.copy()
_G2 = _group_mat(160, 4)
_G2T = _G2.T.copy()

_LS = np.zeros((GRID, IMG), np.float32)
for _h in range(IMG):
    _LS[_h // PATCH, _h] = 1.0
_RS = np.zeros((IMG * 4, GRID * 4), np.float32)
for _w in range(IMG):
    for _c in range(4):
        _RS[4 * _w + _c, (_w // PATCH) * 4 + _c] = 1.0

_SELT = np.zeros((8, 128), np.float32)
_SEL0 = np.zeros((128, 8), np.float32)
_SH1 = np.zeros((128, 128), np.float32)
_SLG = np.zeros((128, 16), np.float32)
for _k in range(8):
    _SELT[_k, 16 * _k] = 1.0
    _SEL0[16 * _k, _k] = 1.0
    _SH1[16 * _k + 1, 16 * _k] = 1.0
    _SLG[16 * _k, 2 * _k] = 1.0
    _SLG[16 * _k + 1, 2 * _k + 1] = 1.0

_MESH = dict(core_axis_name="c", subcore_axis_name="s",
             num_cores=2, num_subcores=16)
_NW = 32


_TPW = N_NODES // _NW
_IDROWS = 1024

def _tok_body(itab, mtab, ids, xout, idsv, ibuf, rbuf, xbuf, sem):
    c = lax.axis_index("c")
    sid = lax.axis_index("s")
    w = sid * 2 + c
    shift = w // 8
    sy = shift // 2
    col0 = 32 * sy + shift
    icol0 = 8 * sy
    pltpu.sync_copy(ids.at[pl.ds(w * 32, 32)], idsv)
    iota16 = lax.broadcasted_iota(I32, (16,), 0)

    def chunk(cidx, carry):
        cps = []
        for j in range(8):
            cps.append(pltpu.async_copy(mtab.at[idsv.at[cidx * 8 + j]],
                                        rbuf.at[j], sem))
            cps.append(pltpu.async_copy(itab.at[idsv.at[cidx * 8 + j]],
                                        ibuf.at[j], sem))
        for cp in cps:
            cp.wait()

        def tok(lt, carry2):
            blk = lt // 16
            lt16 = lt % 16
            ja = jnp.full((16,), blk, I32)
            for k in range(4):
                flat = iota16 + (16 * k)
                rowi = lt16 * 8 + (flat >> 3)
                mval = plsc.load_gather(rbuf, [ja, rowi,
                                               col0 + 4 * (flat & 7)])
                ival = plsc.load_gather(ibuf, [ja, rowi,
                                               icol0 + (flat & 7)])
                xbuf[lt, pl.ds(16 * k, 16)] = ival * mval
            return carry2

        lax.fori_loop(0, 128, tok, 0)
        pltpu.sync_copy(xbuf, xout.at[pl.ds(w * _TPW + cidx * 128, 128)])
        return carry

    lax.fori_loop(0, 4, chunk, 0)


def _sc_tokenize(img_tab, msk_tab, ids2d):
    mesh = plsc.VectorSubcoreMesh(**_MESH)
    fn = pl.kernel(
        _tok_body,
        out_type=jax.ShapeDtypeStruct((N_NODES, 64), F32),
        mesh=mesh,
        scratch_types=[
            pltpu.VMEM((32, 128), I32),
            pltpu.VMEM((8, 128, 16), F32),
            pltpu.VMEM((8, 128, 64), F32),
            pltpu.VMEM((128, 64), F32),
            pltpu.SemaphoreType.DMA,
        ],
        compiler_params=pltpu.CompilerParams(use_tc_tiling_on_sc=False,
                                             needs_layout_passes=False),
    )
    return fn(img_tab, msk_tab, ids2d)


def _psum_body(m4, lm4, lsr, rsb, out):
    u1 = jnp.dot(lsr[...], m4[...], precision=_HI)
    s1 = jnp.dot(u1, rsb[0], precision=_HI)
    u2 = jnp.dot(lsr[...], lm4[...], precision=_HI)
    s2 = jnp.dot(u2, rsb[0], precision=_HI)
    out[0] = jnp.round(s2 / (s1 + EPS))


def _patch_labels(m4, lm4):
    rs3 = jnp.asarray(_RS.reshape(8, 512, 8, 64)[np.arange(8), :,
                                                 np.arange(8)])
    return pl.pallas_call(
        _psum_body,
        grid=(8,),
        in_specs=[
            pl.BlockSpec((IMG, 512), lambda b: (0, b)),
            pl.BlockSpec((IMG, 512), lambda b: (0, b)),
            pl.BlockSpec((GRID, IMG), lambda b: (0, 0)),
            pl.BlockSpec((1, 512, 64), lambda b: (b, 0, 0)),
        ],
        out_specs=pl.BlockSpec((1, GRID, 64), lambda b: (b, 0, 0)),
        out_shape=jax.ShapeDtypeStruct((8, GRID, 64), F32),
    )(m4, lm4, jnp.asarray(_LS), rs3)


_RB_A = 512
_NB_A = N_NODES // _RB_A

def _enc_body(x_ref, m1, b1r, m2, b2r, d1w, d1b, ws, wr, bs, br,
              g1_ref, g1t_ref, g2_ref, g2t_ref,
              t_out, tabs_out, tabr_out):
    g1 = g1_ref[...]
    g1t = g1t_ref[...]
    g2 = g2_ref[...]
    g2t = g2t_ref[...]
    x = x_ref[...]
    y0 = jnp.dot(x, m1[...]) + b1r[...]
    mu = jnp.dot(y0, g1, precision=_HI) * (1.0 / 40.0)
    d = y0 - jnp.dot(mu, g1t, precision=_HI)
    var = jnp.dot(d * d, g1, precision=_HI) * (1.0 / 40.0)
    y = d / jnp.sqrt(jnp.dot(var, g1t, precision=_HI) + 1e-6)
    y = jnp.maximum(y, 0.0)
    z0 = jnp.dot(y, m2[...]) + b2r[...]
    mu2 = jnp.dot(z0, g2, precision=_HI) * (1.0 / 40.0)
    d2 = z0 - jnp.dot(mu2, g2t, precision=_HI)
    var2 = jnp.dot(d2 * d2, g2, precision=_HI) * (1.0 / 40.0)
    z = d2 / jnp.sqrt(jnp.dot(var2, g2t, precision=_HI) + 1e-6)
    z = jnp.maximum(z, 0.0)
    t = jnp.maximum(jnp.dot(z, d1w[...]) + d1b[...], 0.0)
    t_out[...] = t
    tabs_out[...] = jnp.dot(t, ws[...]) + bs[...]
    tabr_out[...] = jnp.dot(t, wr[...]) + br[...]


def _encoder(x_tok, m1, b1r, m2, b2r, d1w, d1b, ws, wr, bs, br):
    row = lambda i: (i, 0)
    full = lambda i: (0, 0)
    return pl.pallas_call(
        _enc_body,
        grid=(_NB_A,),
        in_specs=[
            pl.BlockSpec((_RB_A, 64), row),
            pl.BlockSpec((64, 640), full),
            pl.BlockSpec((1, 640), full),
            pl.BlockSpec((640, 160), full),
            pl.BlockSpec((1, 160), full),
            pl.BlockSpec((160, 16), full),
            pl.BlockSpec((1, 16), full),
            pl.BlockSpec((16, 16), full),
            pl.BlockSpec((16, 16), full),
            pl.BlockSpec((1, 16), full),
            pl.BlockSpec((1, 16), full),
            pl.BlockSpec((640, 16), full),
            pl.BlockSpec((16, 640), full),
            pl.BlockSpec((160, 4), full),
            pl.BlockSpec((4, 160), full),
        ],
        out_specs=[
            pl.BlockSpec((_RB_A, 16), row),
            pl.BlockSpec((_RB_A, 16), row),
            pl.BlockSpec((_RB_A, 16), row),
        ],
        out_shape=[
            jax.ShapeDtypeStruct((N_NODES, 16), F32),
            jax.ShapeDtypeStruct((N_NODES, 16), F32),
            jax.ShapeDtypeStruct((N_NODES, 16), F32),
        ],
    )(x_tok, m1, b1r, m2, b2r, d1w, d1b, ws, wr, bs, br,
      jnp.asarray(_G1), jnp.asarray(_G1T), jnp.asarray(_G2), jnp.asarray(_G2T))


_EPW = N_EDGES // _NW
_CH = 128
_ROWS_PER_W = _EPW // _CH
_GRP = 8
_NROW_E = N_EDGES // 8

def _gather_body(snd, rcv, tabs, tabr, outs, outr,
                 idxs, idxr, bufs, bufr, sem):
    c = lax.axis_index("c")
    s = lax.axis_index("s")
    w = s * 2 + c
    base = w * _ROWS_PER_W
    pltpu.sync_copy(snd.at[pl.ds(base, _ROWS_PER_W)], idxs)
    pltpu.sync_copy(rcv.at[pl.ds(base, _ROWS_PER_W)], idxr)

    def step(k, carry):
        cps = []
        for j in range(_GRP):
            cps.append(pltpu.async_copy(tabs.at[idxs.at[k * _GRP + j]],
                                        bufs.at[j], sem))
            cps.append(pltpu.async_copy(tabr.at[idxr.at[k * _GRP + j]],
                                        bufr.at[j], sem))
        for cp in cps:
            cp.wait()
        pltpu.sync_copy(bufs, outs.at[pl.ds(base + k * _GRP, _GRP)])
        pltpu.sync_copy(bufr, outr.at[pl.ds(base + k * _GRP, _GRP)])
        return carry

    lax.fori_loop(0, _ROWS_PER_W // _GRP, step, 0)


def _edge_gather(snd2d, rcv2d, tabs, tabr):
    mesh = plsc.VectorSubcoreMesh(**_MESH)
    fn = pl.kernel(
        _gather_body,
        out_type=[
            jax.ShapeDtypeStruct((N_EDGES // _CH, _CH, 16), F32),
            jax.ShapeDtypeStruct((N_EDGES // _CH, _CH, 16), F32),
        ],
        mesh=mesh,
        scratch_types=[
            pltpu.VMEM((_ROWS_PER_W, _CH), I32),
            pltpu.VMEM((_ROWS_PER_W, _CH), I32),
            pltpu.VMEM((_GRP, _CH, 16), F32),
            pltpu.VMEM((_GRP, _CH, 16), F32),
            pltpu.SemaphoreType.DMA,
        ],
        compiler_params=pltpu.CompilerParams(use_tc_tiling_on_sc=False),
    )
    return fn(snd2d, rcv2d, tabs, tabr)


_RB_E = 4096
_NB_E = _NROW_E // _RB_E

def _emlp_body(srow, rrow, w2, b2, w3, b3, out):
    h = jnp.maximum(srow[...] + rrow[...], 0.0)
    h = jnp.maximum(jnp.dot(h, w2[...]) + b2[...], 0.0)
    h = jnp.maximum(jnp.dot(h, w3[...]) + b3[...], 0.0)
    out[...] = h


def _edge_mlp(srows, rrows, w2, b2, w3, b3):
    row = lambda i: (i, 0)
    full = lambda i: (0, 0)
    return pl.pallas_call(
        _emlp_body,
        grid=(_NB_E,),
        in_specs=[
            pl.BlockSpec((_RB_E, 128), row),
            pl.BlockSpec((_RB_E, 128), row),
            pl.BlockSpec((128, 128), full),
            pl.BlockSpec((1, 128), full),
            pl.BlockSpec((128, 128), full),
            pl.BlockSpec((1, 128), full),
        ],
        out_specs=pl.BlockSpec((_RB_E, 128), row),
        out_shape=jax.ShapeDtypeStruct((_NROW_E, 128), F32),
    )(srows, rrows, w2, b2, w3, b3)


_ZRB = N_NODES // 16

def _scatter_body(msg, rcv, zeros, out0, out1, agg, mbuf, ridx, sem):
    c = lax.axis_index("c")
    s = lax.axis_index("s")
    w = s * 2 + c
    pltpu.sync_copy(zeros.at[pl.ds(s * _ZRB, _ZRB)],
                    agg.at[pl.ds(s * _ZRB, _ZRB)])
    plsc.subcore_barrier()
    base = w * _ROWS_PER_W
    pltpu.sync_copy(rcv.at[pl.ds(base, _ROWS_PER_W)], ridx)

    def step(k, carry):
        cp = pltpu.async_copy(msg.at[pl.ds(base + k * _GRP, _GRP)], mbuf, sem)
        cp.wait()
        cps = []
        for j in range(_GRP):
            cps.append(pltpu.async_copy(mbuf.at[j],
                                        agg.at[ridx.at[k * _GRP + j]],
                                        sem, add=True))
        for cp2 in cps:
            cp2.wait()
        return carry

    lax.fori_loop(0, _ROWS_PER_W // _GRP, step, 0)
    plsc.subcore_barrier()

    @pl.when(c == 0)
    def _():
        pltpu.sync_copy(agg.at[pl.ds(s * _ZRB, _ZRB)],
                        out0.at[pl.ds(s * _ZRB, _ZRB)])

    @pl.when(c == 1)
    def _():
        pltpu.sync_copy(agg.at[pl.ds(s * _ZRB, _ZRB)],
                        out1.at[pl.ds(s * _ZRB, _ZRB)])


def _edge_scatter(msg2d, rcv2d, zeros):
    mesh = plsc.VectorSubcoreMesh(**_MESH)
    fn = pl.kernel(
        _scatter_body,
        out_type=[
            jax.ShapeDtypeStruct((N_NODES, 16), F32),
            jax.ShapeDtypeStruct((N_NODES, 16), F32),
        ],
        mesh=mesh,
        scratch_types=[
            pltpu.VMEM_SHARED((N_NODES, 16), F32),
            pltpu.VMEM((_GRP, _CH, 16), F32),
            pltpu.VMEM((_ROWS_PER_W, _CH), I32),
            pltpu.SemaphoreType.DMA,
        ],
        compiler_params=pltpu.CompilerParams(use_tc_tiling_on_sc=False),
    )
    return fn(msg2d, rcv2d, zeros)


_NROW_C = N_NODES // 8
_RB_C = 1024
_NB_C = _NROW_C // _RB_C

def _node_body(t, p0, p1, lab8, wt, wa, b1, w2, b2, w3, b3, d2w, d2b,
               selt, sel0, sh1, slg,
               loss_out, logit_out):
    agg = p0[...] + p1[...]
    nf = jnp.maximum(jnp.dot(t[...], wt[...])
                     + jnp.dot(agg, wa[...]) + b1[...], 0.0)
    nf = jnp.maximum(jnp.dot(nf, w2[...]) + b2[...], 0.0)
    nf = jnp.maximum(jnp.dot(nf, w3[...]) + b3[...], 0.0)
    lg = jnp.dot(nf, d2w[...]) + d2b[...]
    l1 = jnp.dot(lg, sh1[...], precision=_HI)
    m = jnp.maximum(lg, l1)
    lse = m + jnp.log(jnp.exp(lg - m) + jnp.exp(l1 - m))
    lp0 = lg - lse
    lp1 = l1 - lse
    lab = jnp.dot(lab8[...], selt[...], precision=_HI)
    loss = -((1.0 - lab) * lp0 + lab * lp1)
    loss_out[...] = jnp.dot(loss, sel0[...], precision=_HI)
    logit_out[...] = jnp.dot(lg, slg[...], precision=_HI)


def _node_mlp(t, p0, p1, lab8, wt, wa, b1, w2, b2, w3, b3, d2w, d2b):
    row = lambda i: (i, 0)
    full = lambda i: (0, 0)
    return pl.pallas_call(
        _node_body,
        grid=(_NB_C,),
        in_specs=[
            pl.BlockSpec((_RB_C, 128), row),
            pl.BlockSpec((_RB_C, 128), row),
            pl.BlockSpec((_RB_C, 128), row),
            pl.BlockSpec((_RB_C, 8), row),
            pl.BlockSpec((128, 128), full),
            pl.BlockSpec((128, 128), full),
            pl.BlockSpec((1, 128), full),
            pl.BlockSpec((128, 128), full),
            pl.BlockSpec((1, 128), full),
            pl.BlockSpec((128, 128), full),
            pl.BlockSpec((1, 128), full),
            pl.BlockSpec((128, 128), full),
            pl.BlockSpec((1, 128), full),
            pl.BlockSpec((8, 128), full),
            pl.BlockSpec((128, 8), full),
            pl.BlockSpec((128, 128), full),
            pl.BlockSpec((128, 16), full),
        ],
        out_specs=[
            pl.BlockSpec((_RB_C, 8), row),
            pl.BlockSpec((_RB_C, 16), row),
        ],
        out_shape=[
            jax.ShapeDtypeStruct((_NROW_C, 8), F32),
            jax.ShapeDtypeStruct((_NROW_C, 16), F32),
        ],
    )(t, p0, p1, lab8, wt, wa, b1, w2, b2, w3, b3, d2w, d2b,
      jnp.asarray(_SELT), jnp.asarray(_SEL0), jnp.asarray(_SH1),
      jnp.asarray(_SLG))


def kernel(curr_image, curr_label, masks, edge_pairs, conv1_w, conv1_b,
           conv2_w, conv2_b, dense1_w, dense1_b, e_w1, e_b1, e_w2, e_b2,
           e_w3, e_b3, n_w1, n_b1, n_w2, n_b2, n_w3, n_b3,
           dense2_w, dense2_b):
    msk4 = masks[0]
    labm4 = curr_label[0] * msk4

    tokv = jnp.arange(N_NODES, dtype=I32)
    sft = tokv >> 12
    syv = sft >> 1
    sxv = sft & 1
    rem = tokv & 4095
    gxv = rem >> 6
    gyv = rem & 63
    ids = ((16 * gxv + 8 * sxv)[:, None]
           + jnp.arange(8, dtype=I32)[None, :]) * 64 + gyv[:, None]
    ids2d = ids.reshape(_IDROWS, 128)

    x_tok = _sc_tokenize(curr_image.reshape(IMG * 64, 16),
                         masks.reshape(IMG * 64, 64), ids2d)

    labp3 = _patch_labels(msk4.reshape(IMG, IMG * 4),
                          labm4.reshape(IMG, IMG * 4))
    labp = labp3.transpose(1, 0, 2).reshape(GRID, GRID * 4)
    lp = labp.reshape(64, 2, 64, 2, 4)
    lp = lp.transpose(3, 1, 0, 2, 4).reshape(4, 4096, 4)
    oh4 = jnp.asarray(np.eye(4, dtype=np.float32))[:, None, :]
    labv = jnp.sum(lp * oh4, axis=-1).reshape(N_NODES)
    labels2 = jnp.stack([1.0 - labv, labv], axis=-1)
    lab8 = labv.reshape(_NROW_C, 8)

    c1 = jnp.dot(jnp.asarray(_S1F).T, conv1_w.reshape(9, 40), precision=_HI)
    m1 = c1.reshape(4, 4, 8, 8, 40).transpose(2, 3, 0, 1, 4).reshape(64, 640)
    c2 = jnp.dot(jnp.asarray(_S2F).T, conv2_w.reshape(9, 40 * 40),
                 precision=_HI)
    m2 = (c2.reshape(2, 2, 4, 4, 40, 40)
          .transpose(2, 3, 4, 0, 1, 5).reshape(640, 160))
    b1_tiled = jnp.tile(conv1_b, 16)[None, :]
    b2_tiled = jnp.tile(conv2_b, 4)[None, :]
    d1w = jnp.pad(dense1_w, ((0, 0), (0, 10)))
    d1b = jnp.pad(dense1_b, (0, 10))[None, :]

    half_b1 = 0.5 * (e_b1 + e_w1[12])
    ws = jnp.pad(e_w1[0:6], ((0, 10), (0, 11)))
    wr = jnp.pad(e_w1[6:12], ((0, 10), (0, 11)))
    bs = jnp.pad(half_b1, (0, 11))[None, :]

    w2p = jnp.pad(e_w2, ((0, 11), (0, 11)))
    b2p = jnp.pad(e_b2, (0, 11))[None, :]
    w3p = jnp.pad(e_w3, ((0, 11), (0, 11)))
    b3p = jnp.pad(e_b3, (0, 11))[None, :]

    eye8 = jnp.asarray(np.eye(8, dtype=np.float32))
    w2bd = jnp.kron(eye8, w2p)
    w3bd = jnp.kron(eye8, w3p)
    b2bd = jnp.tile(b2p, (1, 8))
    b3bd = jnp.tile(b3p, (1, 8))

    wt = jnp.kron(eye8, jnp.pad(n_w1[0:6], ((0, 10), (0, 11))))
    wa = jnp.kron(eye8, jnp.pad(n_w1[6:11], ((0, 11), (0, 11))))
    nb1 = jnp.tile(jnp.pad(n_b1 + n_w1[11], (0, 11)), 8)[None, :]
    nw2 = jnp.kron(eye8, jnp.pad(n_w2, ((0, 11), (0, 11))))
    nb2 = jnp.tile(jnp.pad(n_b2, (0, 11)), 8)[None, :]
    nw3 = jnp.kron(eye8, jnp.pad(n_w3, ((0, 11), (0, 11))))
    nb3 = jnp.tile(jnp.pad(n_b3, (0, 11)), 8)[None, :]
    d2w = jnp.kron(eye8, jnp.pad(dense2_w, ((0, 11), (0, 14))))
    d2b = jnp.tile(jnp.pad(dense2_b, (0, 14)), 8)[None, :]

    t16, tabs, tabr = _encoder(
        x_tok, m1, b1_tiled, m2, b2_tiled, d1w, d1b, ws, wr, bs, bs)

    snd2d = edge_pairs[:, 0].reshape(N_EDGES // _CH, _CH)
    rcv2d = edge_pairs[:, 1].reshape(N_EDGES // _CH, _CH)

    srows, rrows = _edge_gather(snd2d, rcv2d, tabs, tabr)
    msg = _edge_mlp(srows.reshape(_NROW_E, 128), rrows.reshape(_NROW_E, 128),
                    w2bd, b2bd, w3bd, b3bd)

    zeros = jnp.zeros((N_NODES, 16), F32)
    p0, p1 = _edge_scatter(msg.reshape(N_EDGES // _CH, _CH, 16), rcv2d, zeros)

    loss8, logits16 = _node_mlp(
        t16.reshape(_NROW_C, 128), p0.reshape(_NROW_C, 128),
        p1.reshape(_NROW_C, 128), lab8,
        wt, wa, nb1, nw2, nb2, nw3, nb3, d2w, d2b)
    return (loss8.reshape(N_NODES), logits16.reshape(N_NODES, 2), labels2)
